# baseline (speedup 1.0000x reference)
import functools

import jax
import jax.numpy as jnp
import numpy as np
from jax import lax
from jax.experimental import pallas as pl
from jax.experimental.pallas import tpu as pltpu

D_MODEL = 1024
R = 512


def _tc_body(pos_ref, invf_ref, phase_ref, out_ref):
    p = pos_ref[0, 0, :].astype(jnp.float32)
    angle = p[:, None] * invf_ref[0, :][None, :] + phase_ref[0, :][None, :]
    out_ref[...] = jnp.sin(angle)


def _pe_analytic(pos_flat, invf, phase):
    B = pos_flat.shape[0]
    nblk = B // R
    pos3 = pos_flat.reshape(nblk, 1, R)
    return pl.pallas_call(
        _tc_body,
        grid=(nblk,),
        in_specs=[
            pl.BlockSpec((1, 1, R), lambda i: (i, 0, 0)),
            pl.BlockSpec((1, D_MODEL), lambda i: (0, 0)),
            pl.BlockSpec((1, D_MODEL), lambda i: (0, 0)),
        ],
        out_specs=pl.BlockSpec((R, D_MODEL), lambda i: (i, 0)),
        out_shape=jax.ShapeDtypeStruct((B, D_MODEL), jnp.float32),
    )(pos3, invf, phase)


def _consts():
    i = np.arange(D_MODEL, dtype=np.float64) // 2
    invf = np.power(10000.0, -2.0 * i / D_MODEL)
    phase = np.where(np.arange(D_MODEL) % 2 == 1, np.pi / 2.0, 0.0)
    return (jnp.asarray(invf, jnp.float32).reshape(1, D_MODEL),
            jnp.asarray(phase, jnp.float32).reshape(1, D_MODEL))


def kernel(token_positions, PE):
    invf, phase = _consts()
    pos_flat = token_positions.reshape(-1)
    out = _pe_analytic(pos_flat, invf, phase)
    return out.reshape(token_positions.shape + (D_MODEL,))


# speedup vs baseline: 3.8952x; 3.8952x over previous
import functools

import jax
import jax.numpy as jnp
from jax import lax
from jax.experimental import pallas as pl
from jax.experimental.pallas import tpu as pltpu
from jax.experimental.pallas import tpu_sc as plsc

D_MODEL = 1024
NC = 2
NS = 16
NW = NC * NS
K = 8
N_CHUNKS = 128
NBUF = 8
G = 6
W = 2


def _pe_gather(idx3, table):
    B = NW * N_CHUNKS * K
    mesh = plsc.VectorSubcoreMesh(core_axis_name="c", subcore_axis_name="s")

    @functools.partial(
        pl.kernel,
        mesh=mesh,
        out_type=jax.ShapeDtypeStruct((B, D_MODEL), jnp.float32),
        scratch_types=(
            [pltpu.VMEM((N_CHUNKS, K), jnp.int32)]
            + [pltpu.VMEM((K, D_MODEL), jnp.float32) for _ in range(NBUF)]
            + [pltpu.SemaphoreType.DMA for _ in range(2 * NBUF)]
        ),
    )
    def body(idx_hbm, table_hbm, out_hbm, idx_v, *rest):
        bufs = rest[:NBUF]
        gsems = rest[NBUF:2 * NBUF]
        wsems = rest[2 * NBUF:]
        wid = lax.axis_index("s") * NC + lax.axis_index("c")
        base = wid * (N_CHUNKS * K)
        pltpu.sync_copy(idx_hbm.at[wid], idx_v)

        def gather(c, b):
            return pltpu.async_copy(table_hbm.at[idx_v.at[c]], bufs[b], gsems[b])

        def wait_gather(c, b):
            pltpu.make_async_copy(
                table_hbm.at[idx_v.at[c]], bufs[b], gsems[b]).wait()

        def write(c, b):
            return pltpu.async_copy(
                bufs[b], out_hbm.at[pl.ds(base + c * K, K)], wsems[b])

        def wait_write(c, b):
            pltpu.make_async_copy(
                bufs[b], out_hbm.at[pl.ds(base + c * K, K)], wsems[b]).wait()

        for c in range(G):
            gather(c, c % NBUF)

        def step(c, j):
            wait_gather(c, j % NBUF)
            write(c, j % NBUF)

            @pl.when(c >= W)
            def _():
                wait_write(c - W, (j - W) % NBUF)

            @pl.when(c + G < N_CHUNKS)
            def _():
                gather(c + G, (j + G) % NBUF)

        def ring(i, carry):
            for j in range(NBUF):
                step(NBUF * i + j, j)
            return carry

        n_rounds = N_CHUNKS // NBUF
        lax.fori_loop(0, n_rounds, ring, 0)
        for c in range(n_rounds * NBUF, N_CHUNKS):
            step(jnp.int32(c), c % NBUF)
        for c in range(N_CHUNKS - W, N_CHUNKS):
            wait_write(jnp.int32(c), c % NBUF)

    return body(idx3, table)


def kernel(token_positions, PE):
    idx3 = token_positions.reshape(NW, N_CHUNKS, K)
    out = _pe_gather(idx3, PE)
    return out.reshape(token_positions.shape + (D_MODEL,))


# speedup vs baseline: 3.9038x; 1.0022x over previous
import functools

import jax
import jax.numpy as jnp
from jax import lax
from jax.experimental import pallas as pl
from jax.experimental.pallas import tpu as pltpu
from jax.experimental.pallas import tpu_sc as plsc

D_MODEL = 1024
NC = 2
NS = 16
NW = NC * NS
K = 16
N_CHUNKS = 64
NBUF = 4
G = 2
W = 2


def _pe_gather(idx3, table):
    B = NW * N_CHUNKS * K
    mesh = plsc.VectorSubcoreMesh(core_axis_name="c", subcore_axis_name="s")

    @functools.partial(
        pl.kernel,
        mesh=mesh,
        out_type=jax.ShapeDtypeStruct((B, D_MODEL), jnp.float32),
        scratch_types=(
            [pltpu.VMEM((N_CHUNKS, K), jnp.int32)]
            + [pltpu.VMEM((K, D_MODEL), jnp.float32) for _ in range(NBUF)]
            + [pltpu.SemaphoreType.DMA for _ in range(2 * NBUF)]
        ),
    )
    def body(idx_hbm, table_hbm, out_hbm, idx_v, *rest):
        bufs = rest[:NBUF]
        gsems = rest[NBUF:2 * NBUF]
        wsems = rest[2 * NBUF:]
        wid = lax.axis_index("s") * NC + lax.axis_index("c")
        base = wid * (N_CHUNKS * K)
        pltpu.sync_copy(idx_hbm.at[wid], idx_v)

        def gather(c, b):
            return pltpu.async_copy(table_hbm.at[idx_v.at[c]], bufs[b], gsems[b])

        def wait_gather(c, b):
            pltpu.make_async_copy(
                table_hbm.at[idx_v.at[c]], bufs[b], gsems[b]).wait()

        def write(c, b):
            return pltpu.async_copy(
                bufs[b], out_hbm.at[pl.ds(base + c * K, K)], wsems[b])

        def wait_write(c, b):
            pltpu.make_async_copy(
                bufs[b], out_hbm.at[pl.ds(base + c * K, K)], wsems[b]).wait()

        for c in range(G):
            gather(c, c % NBUF)

        def step(c, j):
            wait_gather(c, j % NBUF)
            write(c, j % NBUF)

            @pl.when(c >= W)
            def _():
                wait_write(c - W, (j - W) % NBUF)

            @pl.when(c + G < N_CHUNKS)
            def _():
                gather(c + G, (j + G) % NBUF)

        def ring(i, carry):
            for j in range(NBUF):
                step(NBUF * i + j, j)
            return carry

        n_rounds = N_CHUNKS // NBUF
        lax.fori_loop(0, n_rounds, ring, 0)
        for c in range(n_rounds * NBUF, N_CHUNKS):
            step(jnp.int32(c), c % NBUF)
        for c in range(N_CHUNKS - W, N_CHUNKS):
            wait_write(jnp.int32(c), c % NBUF)

    return body(idx3, table)


def kernel(token_positions, PE):
    idx3 = token_positions.reshape(NW, N_CHUNKS, K)
    out = _pe_gather(idx3, PE)
    return out.reshape(token_positions.shape + (D_MODEL,))
